# Initial kernel scaffold; baseline (speedup 1.0000x reference)
import jax
import jax.numpy as jnp
from jax.experimental import pallas as pl


def kernel(x, liquid_state, params):
    raise NotImplementedError("write your pallas kernel here")


# speedup vs baseline: 2.3917x; 2.3917x over previous
import functools
import math

import jax
import jax.numpy as jnp
from jax.experimental import pallas as pl
from jax.experimental.pallas import tpu as pltpu

F32 = jnp.float32
_SQRT2 = math.sqrt(2.0)


def _gelu(v):
    return 0.5 * v * (1.0 + jax.lax.erf(v / _SQRT2))


def _router_body(xb, liqb, rw1x, rw1l, rb1, rw2, rb2, uw1, ub1, uw2, ub2,
                 w_out, misc, *, nblocks, n_tokens):
    i = pl.program_id(0)
    x_ = xb[...]
    h = (jnp.dot(x_, rw1x[...], preferred_element_type=F32)
         + jnp.dot(liqb[...], rw1l[...], preferred_element_type=F32)
         + rb1[...])
    h = _gelu(h)
    logits = jnp.dot(h, rw2[...], preferred_element_type=F32) + rb2[...]
    m = jnp.max(logits, axis=-1, keepdims=True)
    e = jnp.exp(logits - m)
    p = e / jnp.sum(e, axis=-1, keepdims=True)
    lane = jax.lax.broadcasted_iota(jnp.int32, p.shape, 1)
    p1 = jnp.max(p, axis=-1, keepdims=True)
    i1 = jnp.min(jnp.where(p == p1, lane, 999), axis=-1, keepdims=True)
    pm = jnp.where(lane == i1, -1.0, p)
    p2 = jnp.max(pm, axis=-1, keepdims=True)
    i2 = jnp.min(jnp.where(pm == p2, lane, 999), axis=-1, keepdims=True)
    s12 = p1 + p2
    oh1 = (lane == i1).astype(F32)
    oh2 = (lane == i2).astype(F32)
    w = oh1 * (p1 / s12) + oh2 * (p2 / s12)
    w_out[...] = w

    hu = _gelu(jnp.dot(x_, uw1[...], preferred_element_type=F32) + ub1[...])
    uo = jnp.dot(hu, uw2[...], preferred_element_type=F32) + ub2[...]
    unc = jax.nn.sigmoid(uo[:, 0:1])

    @pl.when(i == 0)
    def _init():
        misc[...] = jnp.zeros_like(misc)

    counts_blk = jnp.sum(oh1 + oh2, axis=0, keepdims=True)
    misc[0:1, :] += counts_blk
    misc[2:3, :] += jnp.full((1, misc.shape[1]), jnp.sum(unc), F32)

    @pl.when(i == nblocks - 1)
    def _fin():
        c = misc[0:1, :]
        el = c / (2.0 * n_tokens)
        lane8 = jax.lax.broadcasted_iota(jnp.int32, el.shape, 1) < 8
        diff = jnp.where(lane8, el - 0.125, 0.0)
        lb = 0.01 * jnp.sum(diff * diff) / 8.0
        misc[0:1, :] = el
        misc[1:2, :] = jnp.full_like(c, lb)
        misc[2:3, :] = misc[2:3, :] / n_tokens


def _expert_body(xb, w1, b1, g, beta, w2, b2, wblk, oin, out, *, col):
    x_ = xb[...]
    h = jnp.dot(x_, w1[...], preferred_element_type=F32) + b1[...]
    mu = jnp.mean(h, axis=-1, keepdims=True)
    var = jnp.mean((h - mu) ** 2, axis=-1, keepdims=True)
    h = (h - mu) * jax.lax.rsqrt(var + 1e-5) * g[...] + beta[...]
    h = _gelu(h)
    eo = jnp.dot(h, w2[...], preferred_element_type=F32) + b2[...]
    wv = wblk[:, col:col + 1]
    out[...] = oin[...] + wv * eo


def kernel(x, liquid_state, params):
    Bsz, Seq, D = x.shape
    N = Bsz * Seq
    L = liquid_state.shape[-1]
    x_flat = x.reshape(N, D)
    liq = jnp.broadcast_to(liquid_state[:, None, :], (Bsz, Seq, L)).reshape(N, L)

    r = params["router"]
    u = params["unc"]
    E = r["W2"].shape[1]
    HR = r["W1"].shape[1]
    HU = u["W1"].shape[1]
    rw1x = r["W1"][:D]
    rw1l = r["W1"][D:]
    rb1 = r["b1"].reshape(1, HR)
    rw2 = jnp.zeros((HR, 128), F32).at[:, :E].set(r["W2"])
    rb2 = jnp.full((1, 128), -1e30, F32).at[0, :E].set(r["b2"])
    uw1 = u["W1"]
    ub1 = u["b1"].reshape(1, HU)
    uw2 = jnp.zeros((HU, 128), F32).at[:, 0:1].set(u["W2"])
    ub2 = jnp.zeros((1, 128), F32).at[0, 0].set(u["b2"][0])

    TB = 512
    NB = N // TB
    full = lambda s: pl.BlockSpec(s, lambda i: (0, 0))
    w_dense, misc = pl.pallas_call(
        functools.partial(_router_body, nblocks=NB, n_tokens=N),
        grid=(NB,),
        in_specs=[
            pl.BlockSpec((TB, D), lambda i: (i, 0)),
            pl.BlockSpec((TB, L), lambda i: (i, 0)),
            full((D, HR)), full((L, HR)), full((1, HR)),
            full((HR, 128)), full((1, 128)),
            full((D, HU)), full((1, HU)),
            full((HU, 128)), full((1, 128)),
        ],
        out_specs=[
            pl.BlockSpec((TB, 128), lambda i: (i, 0)),
            pl.BlockSpec((8, 128), lambda i: (0, 0)),
        ],
        out_shape=[
            jax.ShapeDtypeStruct((N, 128), F32),
            jax.ShapeDtypeStruct((8, 128), F32),
        ],
    )(x_flat, liq, rw1x, rw1l, rb1, rw2, rb2, uw1, ub1, uw2, ub2)

    TBE = 256
    NBE = N // TBE
    out = jnp.zeros((N, D), F32)
    for e in range(E):
        ep = params["experts"][e]
        h = ep["W1"].shape[1]
        out = pl.pallas_call(
            functools.partial(_expert_body, col=e),
            grid=(NBE,),
            in_specs=[
                pl.BlockSpec((TBE, D), lambda i: (i, 0)),
                full((D, h)), full((1, h)), full((1, h)), full((1, h)),
                full((h, D)), full((1, D)),
                pl.BlockSpec((TBE, 128), lambda i: (i, 0)),
                pl.BlockSpec((TBE, D), lambda i: (i, 0)),
            ],
            out_specs=pl.BlockSpec((TBE, D), lambda i: (i, 0)),
            out_shape=jax.ShapeDtypeStruct((N, D), F32),
            input_output_aliases={8: 0},
        )(x_flat, ep["W1"], ep["b1"].reshape(1, h), ep["g"].reshape(1, h),
          ep["beta"].reshape(1, h), ep["W2"], ep["b2"].reshape(1, D),
          w_dense, out)

    output = out.reshape(Bsz, Seq, D)
    expert_loads = misc[0, :E]
    lb_loss = misc[1, 0]
    unc_mean = misc[2, 0]
    return output, lb_loss, expert_loads, unc_mean
